# baseline (speedup 1.0000x reference)
import functools

import jax
import jax.numpy as jnp
import numpy as np
from jax.experimental import pallas as pl
from jax.experimental.pallas import tpu as pltpu

_B, _N, _M, _D = 8, 256, 16, 256
_LOG2E = 1.4426950408889634
_LN2 = 0.6931471805599453

_MASK = ((np.arange(_N)[None, :] == (np.arange(_N * _M)[:, None] // _M))
         .astype(np.float32) * (1.0 / (_M - 1)))


def _loss_kernel(w_ref, mask_ref, e_ref, o_ref):
    n, m, d = _N, _M, _D
    nm = n * m
    alpha = w_ref[0] * _LOG2E

    e4 = e_ref[0]
    e = e4.reshape(nm, d)

    cmat = jnp.sum(e4, axis=1) * (alpha * (1.0 / m))

    d2 = jax.lax.dot_general(
        e, cmat, (((1,), (1,)), ((), ())),
        preferred_element_type=jnp.float32)

    t = jnp.sum(e * e, axis=1, keepdims=True) * alpha

    mk = mask_ref[...]
    dmod = d2 + mk * (d2 - t)

    selfc = jnp.sum(mk * dmod, axis=1, keepdims=True)
    mx = jnp.max(dmod, axis=1, keepdims=True)
    ssum = jnp.sum(jnp.exp2(dmod - mx), axis=1, keepdims=True)
    contrib = mx + jnp.log2(ssum) - (m - 1.0) * selfc
    partial = jnp.sum(contrib) * _LN2
    o_ref[...] = jnp.full((1, 8, 128), partial, jnp.float32)


def _run_local(embeddings, w1):
    bsz, n, m, d = embeddings.shape
    partials = pl.pallas_call(
        _loss_kernel,
        grid=(bsz,),
        in_specs=[
            pl.BlockSpec(memory_space=pltpu.SMEM),
            pl.BlockSpec((n * m, n), lambda i: (0, 0)),
            pl.BlockSpec((1, n, m, d), lambda i: (i, 0, 0, 0)),
        ],
        out_specs=pl.BlockSpec((1, 8, 128), lambda i: (i, 0, 0)),
        out_shape=jax.ShapeDtypeStruct((bsz, 8, 128), jnp.float32),
        compiler_params=pltpu.CompilerParams(
            dimension_semantics=("parallel",),
            vmem_limit_bytes=100 * 1024 * 1024,
        ),
    )(w1, jnp.asarray(_MASK), embeddings)
    return jnp.sum(partials[:, 0, 0])


@functools.partial(jax.jit, static_argnames=())
def kernel(embeddings, w, b):
    del b
    bsz = embeddings.shape[0]
    w1 = jnp.reshape(w.astype(jnp.float32), (1,))
    devs = jax.devices()
    nd = 2 if (len(devs) >= 2 and bsz % 2 == 0) else 1
    if nd == 1:
        return _run_local(embeddings, w1)
    mesh = jax.sharding.Mesh(np.asarray(devs[:nd]), ("x",))
    spec = jax.sharding.PartitionSpec

    def _shard(e, w_):
        return jax.lax.psum(_run_local(e, w_), "x")

    f = jax.shard_map(
        _shard, mesh=mesh,
        in_specs=(spec("x"), spec()),
        out_specs=spec(), check_vma=False)
    return f(embeddings, w1)


# speedup vs baseline: 12.6510x; 12.6510x over previous
import functools

import jax
import jax.numpy as jnp
import numpy as np
from jax.experimental import pallas as pl
from jax.experimental.pallas import tpu as pltpu

_B, _N, _M, _D = 8, 256, 16, 256
_LOG2E = 1.4426950408889634
_LN2 = 0.6931471805599453

_MASK = ((np.arange(_N)[None, :] == (np.arange(_N * _M)[:, None] // _M))
         .astype(np.float32) * (1.0 / (_M - 1)))
_ASEL = ((np.arange(_N)[:, None] == (np.arange(_N * _M)[None, :] // _M))
         .astype(np.float32))


def _loss_kernel(w_ref, mask_ref, asel_ref, e_ref, o_ref):
    n, m, d = _N, _M, _D
    nm = n * m
    alpha = w_ref[0] * _LOG2E

    e = e_ref[0].reshape(nm, d)

    csum = jax.lax.dot_general(
        asel_ref[...], e, (((1,), (0,)), ((), ())),
        preferred_element_type=jnp.float32)
    cmat = csum * (alpha * (1.0 / m))

    d2 = jax.lax.dot_general(
        e, cmat, (((1,), (1,)), ((), ())),
        preferred_element_type=jnp.float32)

    esq = e * e
    t = jnp.sum(esq, axis=1, keepdims=True) * alpha

    dmod = d2 + mask_ref[...] * (d2 - t)

    mx = jnp.max(dmod, axis=1, keepdims=True)
    ssum = jnp.sum(jnp.exp2(dmod - mx), axis=1, keepdims=True)
    lse_total = jnp.sum(mx + jnp.log2(ssum))

    sum_diag = jnp.sum(csum * cmat)
    self_total = (m / (m - 1.0)) * sum_diag - jnp.sum(t) * (1.0 / (m - 1.0))

    partial = (lse_total - self_total) * _LN2
    o_ref[...] = jnp.full((1, 8, 128), partial, jnp.float32)


@functools.partial(jax.jit, static_argnames=())
def kernel(embeddings, w, b):
    del b
    bsz, n, m, d = embeddings.shape
    w1 = jnp.reshape(w.astype(jnp.float32), (1,))
    partials = pl.pallas_call(
        _loss_kernel,
        grid=(bsz,),
        in_specs=[
            pl.BlockSpec(memory_space=pltpu.SMEM),
            pl.BlockSpec((n * m, n), lambda i: (0, 0)),
            pl.BlockSpec((n, n * m), lambda i: (0, 0)),
            pl.BlockSpec((1, n, m, d), lambda i: (i, 0, 0, 0)),
        ],
        out_specs=pl.BlockSpec((1, 8, 128), lambda i: (i, 0, 0)),
        out_shape=jax.ShapeDtypeStruct((bsz, 8, 128), jnp.float32),
        compiler_params=pltpu.CompilerParams(
            dimension_semantics=("parallel",),
            vmem_limit_bytes=100 * 1024 * 1024,
        ),
    )(w1, jnp.asarray(_MASK), jnp.asarray(_ASEL), embeddings)
    return jnp.sum(partials[:, 0, 0])
